# baseline (speedup 1.0000x reference)
import numpy as np
import jax
import jax.numpy as jnp
from jax.experimental import pallas as pl
from jax.experimental.pallas import tpu as pltpu

_BATCH = 512
_IN = 128
_H = 32
_T = 32

_TG = 16
_CB = 128

_ADJ_LIST = [
    [0, 2, 5, 8, 11],
    [0, 1, 4, 7, 10],
    [0, 3, 6, 9, 12, 15],
    [9, 14, 17, 19, 21],
    [9, 13, 16, 18, 20],
]


def _build_a_norm() -> np.ndarray:
    num_nodes = max(max(s) for s in _ADJ_LIST) + 1
    a = np.zeros((num_nodes, num_nodes), dtype=np.float64)
    for sub in _ADJ_LIST:
        for i in range(len(sub)):
            for j in range(i + 1, len(sub)):
                a[sub[i], sub[j]] = 1.0
                a[sub[j], sub[i]] = 1.0
    deg = a.sum(axis=0)
    norm = 1.0 / np.sqrt(np.clip(deg, 1.0, None))
    return (norm[:, None] * a * norm[None, :]).astype(np.float32)


_A_NORM = _build_a_norm()
_N = _A_NORM.shape[0]
_NH = _N * _H
_A_EXP = np.repeat(_A_NORM, _H, axis=1)


def _tile_nodes(v):
    return jnp.concatenate([v] * _N, axis=-1)


def _body(x_ref, wr_ref, br_ref, wz_ref, bz_ref, wh_ref, bh_ref,
          wg_ref, bg_ref, aexp_ref, out_ref, xg, h_ref, k_ref, bgt_ref):
    p = pl.program_id(0)

    @pl.when(p == 0)
    def _init():
        wg_tile = 0.5 * _tile_nodes(wg_ref[...])
        for m in range(_N):
            k_ref[m * _H:(m + 1) * _H, :] = (
                wg_tile * aexp_ref[m:m + 1, :]
            ).astype(jnp.bfloat16)
        bgt_ref[...] = 0.5 * _tile_nodes(bg_ref[...])
        x_val = x_ref[...]
        for j, (w_ref, b_ref, s) in enumerate(
                ((wr_ref, br_ref, 0.5), (wz_ref, bz_ref, 0.5),
                 (wh_ref, bh_ref, 1.0))):
            xg[j] = (
                jnp.dot(x_val, s * _tile_nodes(w_ref[...]),
                        preferred_element_type=jnp.float32)
                + s * _tile_nodes(b_ref[...])
            ).astype(jnp.bfloat16)
        h_ref[...] = jnp.zeros((_BATCH, _NH), jnp.bfloat16)

    k = k_ref[...]
    bgt = bgt_ref[...]

    h = h_ref[...]
    for i in range(_TG):
        gh32 = jnp.dot(h, k, preferred_element_type=jnp.float32)
        gh = (gh32 + bgt).astype(jnp.bfloat16)
        t_r = jnp.tanh(xg[0] + gh)
        t_z = jnp.tanh(xg[1] + gh)
        h_tilde = jnp.tanh(xg[2] + gh + t_r * gh)
        s = h_tilde - h
        h = h + 0.5 * (s + t_z * s)
        out_ref[:, i * _NH:(i + 1) * _NH] = h.astype(jnp.float32)
    h_ref[...] = h


def kernel(x, Wr, br, Wz, bz, Wh, bh, Wg, bg):
    full = lambda *s: pl.BlockSpec(s, lambda p: (0,) * len(s))
    out = pl.pallas_call(
        _body,
        grid=(_T // _TG,),
        in_specs=[
            full(_BATCH, _IN),
            full(_IN, _H),
            full(1, _H),
            full(_IN, _H),
            full(1, _H),
            full(_IN, _H),
            full(1, _H),
            full(_H, _H),
            full(1, _H),
            full(_N, _NH),
        ],
        out_specs=pl.BlockSpec((_BATCH, _TG * _NH), lambda p: (0, p)),
        out_shape=jax.ShapeDtypeStruct((_BATCH, _T * _NH), jnp.float32),
        scratch_shapes=[
            pltpu.VMEM((3, _BATCH, _NH), jnp.bfloat16),
            pltpu.VMEM((_BATCH, _NH), jnp.bfloat16),
            pltpu.VMEM((_NH, _NH), jnp.bfloat16),
            pltpu.VMEM((1, _NH), jnp.float32),
        ],
        compiler_params=pltpu.CompilerParams(
            dimension_semantics=("arbitrary",),
        ),
    )(x, Wr, br.reshape(1, _H), Wz, bz.reshape(1, _H),
      Wh, bh.reshape(1, _H), Wg, bg.reshape(1, _H), jnp.asarray(_A_EXP))
    return out


# speedup vs baseline: 1.0404x; 1.0404x over previous
import numpy as np
import jax
import jax.numpy as jnp
from jax.experimental import pallas as pl
from jax.experimental.pallas import tpu as pltpu

_BATCH = 512
_IN = 128
_H = 32
_T = 32

_TG = 8
_CB = 128

_ADJ_LIST = [
    [0, 2, 5, 8, 11],
    [0, 1, 4, 7, 10],
    [0, 3, 6, 9, 12, 15],
    [9, 14, 17, 19, 21],
    [9, 13, 16, 18, 20],
]


def _build_a_norm() -> np.ndarray:
    num_nodes = max(max(s) for s in _ADJ_LIST) + 1
    a = np.zeros((num_nodes, num_nodes), dtype=np.float64)
    for sub in _ADJ_LIST:
        for i in range(len(sub)):
            for j in range(i + 1, len(sub)):
                a[sub[i], sub[j]] = 1.0
                a[sub[j], sub[i]] = 1.0
    deg = a.sum(axis=0)
    norm = 1.0 / np.sqrt(np.clip(deg, 1.0, None))
    return (norm[:, None] * a * norm[None, :]).astype(np.float32)


_A_NORM = _build_a_norm()
_N = _A_NORM.shape[0]
_NH = _N * _H
_A_EXP = np.repeat(_A_NORM, _H, axis=1)


def _tile_nodes(v):
    return jnp.concatenate([v] * _N, axis=-1)


def _body(x_ref, wr_ref, br_ref, wz_ref, bz_ref, wh_ref, bh_ref,
          wg_ref, bg_ref, aexp_ref, out_ref, xg, h_ref, k_ref, bgt_ref):
    p = pl.program_id(0)

    @pl.when(p == 0)
    def _init():
        wg_tile = 0.5 * _tile_nodes(wg_ref[...])
        for m in range(_N):
            k_ref[m * _H:(m + 1) * _H, :] = (
                wg_tile * aexp_ref[m:m + 1, :]
            ).astype(jnp.bfloat16)
        bgt_ref[...] = 0.5 * _tile_nodes(bg_ref[...])[None, :]
        x_val = x_ref[...]
        for j, (w_ref, b_ref, s) in enumerate(
                ((wr_ref, br_ref, 0.5), (wz_ref, bz_ref, 0.5),
                 (wh_ref, bh_ref, 1.0))):
            xg[j] = (
                jnp.dot(x_val, s * _tile_nodes(w_ref[...]),
                        preferred_element_type=jnp.float32)
                + s * _tile_nodes(b_ref[...])[None, :]
            ).astype(jnp.bfloat16)
        h_ref[...] = jnp.zeros((_BATCH, _NH), jnp.bfloat16)

    k = k_ref[...]
    bgt = bgt_ref[...]

    h = h_ref[...]
    for i in range(_TG):
        gh32 = jnp.dot(h, k, preferred_element_type=jnp.float32)
        gh = (gh32 + bgt).astype(jnp.bfloat16)
        t_r = jnp.tanh(xg[0] + gh)
        t_z = jnp.tanh(xg[1] + gh)
        h_tilde = jnp.tanh(xg[2] + gh + t_r * gh)
        s = h_tilde - h
        h = h + 0.5 * (s + t_z * s)
        out_ref[:, i * _NH:(i + 1) * _NH] = h.astype(jnp.float32)
    h_ref[...] = h


def kernel(x, Wr, br, Wz, bz, Wh, bh, Wg, bg):
    full = lambda *s: pl.BlockSpec(s, lambda p: (0,) * len(s))
    out = pl.pallas_call(
        _body,
        grid=(_T // _TG,),
        in_specs=[
            full(_BATCH, _IN),
            full(_IN, _H),
            full(_H),
            full(_IN, _H),
            full(_H),
            full(_IN, _H),
            full(_H),
            full(_H, _H),
            full(_H),
            full(_N, _NH),
        ],
        out_specs=pl.BlockSpec((_BATCH, _TG * _NH), lambda p: (0, p)),
        out_shape=jax.ShapeDtypeStruct((_BATCH, _T * _NH), jnp.float32),
        scratch_shapes=[
            pltpu.VMEM((3, _BATCH, _NH), jnp.bfloat16),
            pltpu.VMEM((_BATCH, _NH), jnp.bfloat16),
            pltpu.VMEM((_NH, _NH), jnp.bfloat16),
            pltpu.VMEM((1, _NH), jnp.float32),
        ],
        compiler_params=pltpu.CompilerParams(
            dimension_semantics=("arbitrary",),
        ),
    )(x, Wr, br, Wz, bz, Wh, bh, Wg, bg, jnp.asarray(_A_EXP))
    return out


# speedup vs baseline: 1.0556x; 1.0146x over previous
import numpy as np
import jax
import jax.numpy as jnp
from jax.experimental import pallas as pl
from jax.experimental.pallas import tpu as pltpu

_BATCH = 512
_IN = 128
_H = 32
_T = 32

_TG = 8
_CB = 128

_ADJ_LIST = [
    [0, 2, 5, 8, 11],
    [0, 1, 4, 7, 10],
    [0, 3, 6, 9, 12, 15],
    [9, 14, 17, 19, 21],
    [9, 13, 16, 18, 20],
]


def _build_a_norm() -> np.ndarray:
    num_nodes = max(max(s) for s in _ADJ_LIST) + 1
    a = np.zeros((num_nodes, num_nodes), dtype=np.float64)
    for sub in _ADJ_LIST:
        for i in range(len(sub)):
            for j in range(i + 1, len(sub)):
                a[sub[i], sub[j]] = 1.0
                a[sub[j], sub[i]] = 1.0
    deg = a.sum(axis=0)
    norm = 1.0 / np.sqrt(np.clip(deg, 1.0, None))
    return (norm[:, None] * a * norm[None, :]).astype(np.float32)


_A_NORM = _build_a_norm()
_N = _A_NORM.shape[0]
_NH = _N * _H
_A_EXP = np.repeat(_A_NORM, _H, axis=1)


def _tile_nodes(v):
    return jnp.concatenate([v] * _N, axis=-1)


def _body(x_ref, wr_ref, br_ref, wz_ref, bz_ref, wh_ref, bh_ref,
          wg_ref, bg_ref, aexp_ref, out_ref, xg, h_ref, k_ref, bgt_ref):
    p = pl.program_id(0)

    @pl.when(p == 0)
    def _init():
        wg_tile = 0.5 * _tile_nodes(wg_ref[...])
        for m in range(_N):
            k_ref[m * _H:(m + 1) * _H, :] = (
                wg_tile * aexp_ref[m:m + 1, :]
            ).astype(jnp.bfloat16)
        bgt_ref[...] = 0.5 * _tile_nodes(bg_ref[...])
        x_val = x_ref[...]
        for j, (w_ref, b_ref, s) in enumerate(
                ((wr_ref, br_ref, 0.5), (wz_ref, bz_ref, 0.5),
                 (wh_ref, bh_ref, 1.0))):
            xg[j] = (
                jnp.dot(x_val, s * _tile_nodes(w_ref[...]),
                        preferred_element_type=jnp.float32)
                + s * _tile_nodes(b_ref[...])
            ).astype(jnp.bfloat16)
        h_ref[...] = jnp.zeros((_BATCH, _NH), jnp.bfloat16)

    k = k_ref[...]
    bgt = bgt_ref[...]

    h = h_ref[...]
    for i in range(_TG):
        gh32 = jnp.dot(h, k, preferred_element_type=jnp.float32)
        gh = (gh32 + bgt).astype(jnp.bfloat16)
        t_r = jnp.tanh(xg[0] + gh)
        t_z = jnp.tanh(xg[1] + gh)
        h_tilde = jnp.tanh(xg[2] + gh + t_r * gh)
        s = h_tilde - h
        h = h + 0.5 * (s + t_z * s)
        out_ref[:, i * _NH:(i + 1) * _NH] = h.astype(jnp.float32)
    h_ref[...] = h


def kernel(x, Wr, br, Wz, bz, Wh, bh, Wg, bg):
    full = lambda *s: pl.BlockSpec(s, lambda p: (0,) * len(s))
    out = pl.pallas_call(
        _body,
        grid=(_T // _TG,),
        in_specs=[
            full(_BATCH, _IN),
            full(_IN, _H),
            full(1, _H),
            full(_IN, _H),
            full(1, _H),
            full(_IN, _H),
            full(1, _H),
            full(_H, _H),
            full(1, _H),
            full(_N, _NH),
        ],
        out_specs=pl.BlockSpec((_BATCH, _TG * _NH), lambda p: (0, p)),
        out_shape=jax.ShapeDtypeStruct((_BATCH, _T * _NH), jnp.float32),
        scratch_shapes=[
            pltpu.VMEM((3, _BATCH, _NH), jnp.bfloat16),
            pltpu.VMEM((_BATCH, _NH), jnp.bfloat16),
            pltpu.VMEM((_NH, _NH), jnp.bfloat16),
            pltpu.VMEM((1, _NH), jnp.float32),
        ],
        compiler_params=pltpu.CompilerParams(
            dimension_semantics=("arbitrary",),
        ),
    )(x, Wr, br.reshape(1, _H), Wz, bz.reshape(1, _H),
      Wh, bh.reshape(1, _H), Wg, bg.reshape(1, _H), jnp.asarray(_A_EXP))
    return out
